# baseline (speedup 1.0000x reference)
import functools

import jax
import jax.numpy as jnp
from jax import lax
from jax.experimental import pallas as pl
from jax.experimental.pallas import tpu as pltpu
from jax.experimental.pallas import tpu_sc as plsc

NUM_BINS = 512
T = 32
TBL = NUM_BINS * NUM_BINS
B = 16384
NC, NS, L = 2, 16, 16
NW = NC * NS
NBW = B // NW
SUBL = 8
LANE = 128
CTILES = TBL // LANE
NCH = 2
CHB = NBW // NCH
CHV = CHB // L
CHW = T * CHB


def _sc_tile_code(sv, wp, wv, wr):
    mesh = plsc.VectorSubcoreMesh(
        core_axis_name="c", subcore_axis_name="s",
        num_cores=NC, num_subcores=NS)

    @functools.partial(
        pl.kernel,
        out_type=jax.ShapeDtypeStruct((4 * B,), jnp.float32),
        mesh=mesh,
        scratch_types=[
            pltpu.VMEM((2 * NBW,), jnp.float32),
            pltpu.VMEM((T * NBW,), jnp.int32),
            pltpu.VMEM((T * NBW,), jnp.float32),
            pltpu.VMEM((T * NBW,), jnp.float32),
            pltpu.VMEM((T * NBW,), jnp.float32),
            pltpu.VMEM((4 * NBW,), jnp.float32),
            [pltpu.SemaphoreType.DMA] * 6,
        ],
    )
    def k(sv_hbm, wp_hbm, wv_hbm, wr_hbm, out_hbm,
          sv_v, idx_v, gp_v, gv_v, gr_v, ov_v, sems):
        wid = lax.axis_index("s") * NC + lax.axis_index("c")
        pltpu.sync_copy(sv_hbm.at[pl.ds(wid * (2 * NBW), 2 * NBW)], sv_v)

        def idx_chunk(ch):
            def body(vb, _):
                e0 = ch * CHB + vb * L
                blk = (e0 // LANE) * (2 * LANE) + (e0 % LANE)
                s0 = sv_v[pl.ds(blk, L)] * 512.0
                s1 = sv_v[pl.ds(blk + LANE, L)] * 512.0
                for t in range(T):
                    c = float(t) / 32.0
                    i0 = jnp.minimum((s0 + c).astype(jnp.int32), NUM_BINS - 1)
                    i1 = jnp.minimum((s1 + c).astype(jnp.int32), NUM_BINS - 1)
                    f = i0 * NUM_BINS + i1
                    tconst = (t // SUBL) * (CTILES * SUBL * LANE) + (t % SUBL) * LANE
                    idx_v[pl.ds(ch * CHW + t * CHB + vb * L, L)] = (
                        ((f >> 7) << 10) + (f & (LANE - 1)) + tconst)
                return 0
            lax.fori_loop(0, CHV, body, 0)

        def fire(ch):
            s = ch * CHW
            par = 3 * (ch % 2)
            return (
                pltpu.async_copy(wp_hbm.at[idx_v.at[pl.ds(s, CHW)]],
                                 gp_v.at[pl.ds(s, CHW)], sems[par + 0]),
                pltpu.async_copy(wv_hbm.at[idx_v.at[pl.ds(s, CHW)]],
                                 gv_v.at[pl.ds(s, CHW)], sems[par + 1]),
                pltpu.async_copy(wr_hbm.at[idx_v.at[pl.ds(s, CHW)]],
                                 gr_v.at[pl.ds(s, CHW)], sems[par + 2]),
            )

        def acc_chunk(ch):
            def body(vb, _):
                e0 = ch * CHB + vb * L
                blk = (e0 // LANE) * (2 * LANE) + (e0 % LANE)
                ob = (e0 // LANE) * (4 * LANE) + (e0 % LANE)
                ap = jnp.zeros((L,), jnp.float32)
                av = jnp.zeros((L,), jnp.float32)
                ar = jnp.zeros((L,), jnp.float32)
                for t in range(T):
                    s = ch * CHW + t * CHB + vb * L
                    ap = ap + gp_v[pl.ds(s, L)]
                    av = av + gv_v[pl.ds(s, L)]
                    ar = ar + gr_v[pl.ds(s, L)]
                c0 = sv_v[pl.ds(blk, L)]
                c1 = sv_v[pl.ds(blk + LANE, L)]
                ov_v[pl.ds(ob, L)] = jnp.clip(c0 + ap, 0.0, 1.0)
                ov_v[pl.ds(ob + LANE, L)] = jnp.clip(c1 + av, 0.0, 1.0)
                ov_v[pl.ds(ob + 2 * LANE, L)] = ar
                return 0
            lax.fori_loop(0, CHV, body, 0)

        inflight = []
        for ch in range(NCH):
            idx_chunk(ch)
            cps = fire(ch)
            inflight.append(cps)
            if ch >= 1:
                for c in inflight[ch - 1]:
                    c.wait()
                acc_chunk(ch - 1)
        for c in inflight[NCH - 1]:
            c.wait()
        acc_chunk(NCH - 1)

        pltpu.sync_copy(ov_v, out_hbm.at[pl.ds(wid * (4 * NBW), 4 * NBW)])

    return k(sv, wp, wv, wr)


def _phys_flat(w):
    return (w.reshape(T // SUBL, SUBL, CTILES, LANE)
             .transpose(0, 2, 1, 3)
             .reshape(-1))


def kernel(state, w_p, w_v, w_r):
    sv = state.reshape(B // LANE, LANE, 2).transpose(0, 2, 1).reshape(-1)
    flat = _sc_tile_code(sv, _phys_flat(w_p), _phys_flat(w_v), _phys_flat(w_r))
    out4 = flat.reshape(B // LANE, 4, LANE).transpose(0, 2, 1).reshape(B, 4)
    return out4[:, :3]
